# baseline (speedup 1.0000x reference)
import functools

import jax
import jax.numpy as jnp
from jax import lax
from jax.experimental import pallas as pl
from jax.experimental.pallas import tpu as pltpu
from jax.experimental.pallas import tpu_sc as plsc

B, P, C = 32, 20000, 21
NC, NS, L = 2, 16, 16
CHUNK = 2000
NCHUNK = P // CHUNK
GROUPS = CHUNK // L
LOC_VECS = CHUNK * 4 // L
KEY_VECS = P // L
SIGN = -2147483648

_LN2 = 0.6931471805599453


def _ln(s):
    bits = plsc.bitcast(s, jnp.int32)
    e = (bits >> 23) - 127
    m = plsc.bitcast((bits & 0x007FFFFF) | 0x3F800000, jnp.int32)
    m = plsc.bitcast(m, jnp.float32)
    big = m > 1.4142135381698608
    m = jnp.where(big, m * 0.5, m)
    e = jnp.where(big, e + 1, e)
    t = (m - 1.0) / (m + 1.0)
    t2 = t * t
    p = 1.0 + t2 * (1.0 / 3.0 + t2 * (1.0 / 5.0 + t2 * (1.0 / 7.0)))
    return 2.0 * t * p + e.astype(jnp.float32) * _LN2


def _sc_body(conf_hbm, targ_hbm, locp_hbm, loct_hbm, out_hbm,
             conf_v, targ_v, confl_v, keys_v, locp_v, loct_v, out_v):
    cid = lax.axis_index("c")
    sid = lax.axis_index("s")
    b = sid * NC + cid
    lanes = lax.iota(jnp.int32, 16)

    pltpu.sync_copy(targ_hbm.at[pl.ds(b * P, P)], targ_v)

    def chunk_body(j, carry):
        npos_acc, lacc = carry
        pltpu.sync_copy(conf_hbm.at[pl.ds(b * (P * C) + j * (CHUNK * C),
                                          CHUNK * C)], conf_v)
        pltpu.sync_copy(locp_hbm.at[pl.ds(b * (P * 4) + j * (CHUNK * 4),
                                          CHUNK * 4)], locp_v)
        pltpu.sync_copy(loct_hbm.at[pl.ds(b * (P * 4) + j * (CHUNK * 4),
                                          CHUNK * 4)], loct_v)

        def group_body(g, npos):
            row_p = j * CHUNK + g * L
            t = targ_v[pl.ds(row_p, L)]
            base = (g * L + lanes) * C
            vals = [plsc.load_gather(conf_v, [base + c]) for c in range(C)]
            m = functools.reduce(jnp.maximum, vals)
            s = functools.reduce(
                jnp.add, [jnp.exp(v - m) for v in vals])
            lse = _ln(s) + m
            gathered = plsc.load_gather(conf_v, [base + t])
            conf_l = lse - gathered
            ispos = t > 0
            cln = jnp.where(ispos, 0.0, conf_l)
            kbits = plsc.bitcast(cln, jnp.int32)
            key = jnp.where(kbits >= 0, kbits, SIGN - kbits)
            confl_v[pl.ds(row_p, L)] = conf_l
            keys_v[pl.ds(row_p, L)] = key
            return npos + jnp.where(ispos, 1, 0)

        npos_acc = lax.fori_loop(0, GROUPS, group_body, npos_acc)

        def loc_body(i, acc):
            off = i * L
            d = locp_v[pl.ds(off, L)] - loct_v[pl.ds(off, L)]
            a = jnp.abs(d)
            sl1 = jnp.where(a < 1.0, 0.5 * d * d, a - 0.5)
            prior = (j * (CHUNK * 4) + off + lanes) >> 2
            tl = plsc.load_gather(targ_v, [prior])
            return acc + jnp.where(tl > 0, sl1, 0.0)

        lacc = lax.fori_loop(0, LOC_VECS, loc_body, lacc)
        return npos_acc, lacc

    npos_acc, lacc = lax.fori_loop(
        0, NCHUNK, chunk_body,
        (jnp.zeros((L,), jnp.int32), jnp.zeros((L,), jnp.float32)))

    npos = jnp.sum(npos_acc)
    k = jnp.minimum(3 * npos, P - 1)

    def bit_body(i, ubits):
        bit = jnp.int32(1) << (31 - i)
        cand_key = (ubits | (bit - 1)) ^ SIGN

        def count_body(v, acc):
            kv = keys_v[pl.ds(v * L, L)]
            return acc + jnp.where(kv > cand_key, 1, 0)

        cnt = jnp.sum(lax.fori_loop(0, KEY_VECS, count_body,
                                    jnp.zeros((L,), jnp.int32)))
        return jnp.where(cnt <= k, ubits, ubits | bit)

    ubits = lax.fori_loop(0, 32, bit_body, jnp.int32(0))
    pivot_key = ubits ^ SIGN

    def final_body(v, acc):
        sl = pl.ds(v * L, L)
        mask = (targ_v[sl] > 0) | (keys_v[sl] > pivot_key)
        return acc + jnp.where(mask, confl_v[sl], 0.0)

    cacc = lax.fori_loop(0, KEY_VECS, final_body, jnp.zeros((L,), jnp.float32))

    loc_s = jnp.sum(lacc)
    conf_s = jnp.sum(cacc)
    npos_f = npos.astype(jnp.float32)
    out_v[...] = jnp.where(lanes == 0, loc_s,
                           jnp.where(lanes == 1, conf_s,
                                     jnp.where(lanes == 2, npos_f, 0.0)))
    pltpu.sync_copy(out_v, out_hbm.at[b])


_sc_kernel = pl.kernel(
    _sc_body,
    out_type=jax.ShapeDtypeStruct((B, L), jnp.float32),
    mesh=plsc.VectorSubcoreMesh(core_axis_name="c", subcore_axis_name="s"),
    compiler_params=pltpu.CompilerParams(needs_layout_passes=False),
    scratch_types=[
        pltpu.VMEM((CHUNK * C,), jnp.float32),
        pltpu.VMEM((P,), jnp.int32),
        pltpu.VMEM((P,), jnp.float32),
        pltpu.VMEM((P,), jnp.int32),
        pltpu.VMEM((CHUNK * 4,), jnp.float32),
        pltpu.VMEM((CHUNK * 4,), jnp.float32),
        pltpu.VMEM((L,), jnp.float32),
    ],
)


def _finish_body(p_ref, o_ref):
    x = p_ref[...]
    lane = lax.broadcasted_iota(jnp.int32, (B, L), 1)
    loc_s = jnp.sum(jnp.where(lane == 0, x, 0.0))
    conf_s = jnp.sum(jnp.where(lane == 1, x, 0.0))
    npos = jnp.sum(jnp.where(lane == 2, x, 0.0))
    loss = (loc_s + conf_s) / jnp.maximum(npos, 1.0)
    o_ref[...] = jnp.full((1, 1), loss, jnp.float32)


_finish = pl.pallas_call(
    _finish_body,
    out_shape=jax.ShapeDtypeStruct((1, 1), jnp.float32),
)


def kernel(loc_preds, conf_preds, loc_targets, conf_targets):
    ct = conf_targets.astype(jnp.int32).reshape(-1)
    partials = _sc_kernel(conf_preds.reshape(-1), ct,
                          loc_preds.reshape(-1), loc_targets.reshape(-1))
    return _finish(partials)[0, 0]
